# baseline (speedup 1.0000x reference)
import functools

import jax
import jax.numpy as jnp
from jax import lax
from jax.experimental import pallas as pl
from jax.experimental.pallas import tpu as pltpu
from jax.experimental.pallas import tpu_sc as plsc


def _sc_gather(centers, W_in):
    B, = centers.shape
    V, D = W_in.shape
    info = plsc.get_sparse_core_info()
    NC, NS = info.num_cores, info.num_subcores
    NW = NC * NS
    b_per_w = B // NW
    mesh = plsc.VectorSubcoreMesh(core_axis_name="c", subcore_axis_name="s")

    @functools.partial(
        pl.kernel,
        mesh=mesh,
        out_type=jax.ShapeDtypeStruct((B, D), jnp.float32),
        scratch_types=[
            pltpu.VMEM((b_per_w,), jnp.int32),
            pltpu.VMEM((b_per_w, D), jnp.float32),
            pltpu.SemaphoreType.DMA,
        ],
    )
    def gather_kernel(idx_hbm, table_hbm, out_hbm, idx_v, rows_v, sem):
        wid = lax.axis_index("s") * NC + lax.axis_index("c")
        base = wid * b_per_w
        pltpu.sync_copy(idx_hbm.at[pl.ds(base, b_per_w)], idx_v)
        pltpu.async_copy(table_hbm.at[idx_v], rows_v, sem).wait()
        pltpu.sync_copy(rows_v, out_hbm.at[pl.ds(base, b_per_w)])

    return gather_kernel(centers, W_in)


def _tc_project(emb, W_out_bf16, b_out):
    B, D = emb.shape
    V = W_out_bf16.shape[1]
    MB = 32
    NSTEP = B // MB
    NBUF = 2
    b2 = b_out.reshape(1, V)

    def body(emb_ref, w_ref, b_ref, out_hbm, buf, sem):
        j = pl.program_id(0)
        slot = jax.lax.rem(j, NBUF)

        @pl.when(j >= NBUF)
        def _wait_ring():
            pj = j - NBUF
            pltpu.make_async_copy(
                buf.at[slot],
                out_hbm.at[pl.ds(pj * MB, MB), :],
                sem.at[slot],
            ).wait()

        buf[slot] = (
            jnp.dot(emb_ref[...].astype(jnp.bfloat16), w_ref[...],
                    preferred_element_type=jnp.float32)
            + b_ref[...]
        )
        pltpu.make_async_copy(
            buf.at[slot],
            out_hbm.at[pl.ds(j * MB, MB), :],
            sem.at[slot],
        ).start()

        @pl.when(j == NSTEP - 1)
        def _drain():
            for pj in range(max(0, NSTEP - NBUF), NSTEP):
                pslot = pj % NBUF
                pltpu.make_async_copy(
                    buf.at[pslot],
                    out_hbm.at[pl.ds(pj * MB, MB), :],
                    sem.at[pslot],
                ).wait()

    return pl.pallas_call(
        body,
        grid=(NSTEP,),
        in_specs=[
            pl.BlockSpec((MB, D), lambda j: (j, 0)),
            pl.BlockSpec((D, V), lambda j: (0, 0)),
            pl.BlockSpec((1, V), lambda j: (0, 0)),
        ],
        out_specs=pl.BlockSpec(memory_space=pl.ANY),
        out_shape=jax.ShapeDtypeStruct((B, V), jnp.float32),
        scratch_shapes=[
            pltpu.VMEM((NBUF, MB, V), jnp.float32),
            pltpu.SemaphoreType.DMA((NBUF,)),
        ],
        compiler_params=pltpu.CompilerParams(
            dimension_semantics=("arbitrary",),
            vmem_limit_bytes=100 * 1024 * 1024,
        ),
    )(emb, W_out_bf16, b2)


def kernel(centers, W_in, W_out, b_out):
    emb = _sc_gather(centers.astype(jnp.int32), W_in)
    return _tc_project(emb, W_out.astype(jnp.bfloat16), b_out)
